# baseline (speedup 1.0000x reference)
import functools

import jax
import jax.numpy as jnp
from jax import lax
from jax.experimental import pallas as pl
from jax.experimental.pallas import tpu as pltpu
from jax.experimental.pallas import tpu_sc as plsc

NC, NS = 2, 16
NW = NC * NS
CHUNK = 128
NBUF = 3


def _make_gather_plane(B, D):
    assert B % (NW * CHUNK) == 0
    bpw = B // NW
    nchunks = bpw // CHUNK

    mesh = plsc.VectorSubcoreMesh(
        core_axis_name="c", subcore_axis_name="s",
        num_cores=NC, num_subcores=NS)

    @functools.partial(
        pl.kernel,
        out_type=jax.ShapeDtypeStruct((B, D), jnp.float32),
        mesh=mesh,
        scratch_types=[
            pltpu.VMEM((bpw,), jnp.int32),
            pltpu.VMEM((NBUF, CHUNK, D), jnp.float32),
        ] + [pltpu.SemaphoreType.DMA] * (2 * NBUF),
    )
    def k(idx_hbm, w_hbm, out_hbm, idx_v, rows_v, *sems):
        gsem = sems[0:NBUF]
        ssem = sems[NBUF:2 * NBUF]
        wid = lax.axis_index("s") * NC + lax.axis_index("c")
        base = wid * bpw
        pltpu.sync_copy(idx_hbm.at[pl.ds(base, bpw)], idx_v)

        def gather_start(c, b):
            pltpu.async_copy(
                w_hbm.at[idx_v.at[pl.ds(c * CHUNK, CHUNK)]],
                rows_v.at[b], gsem[b])

        def gather_wait(b):
            pltpu.make_async_copy(
                w_hbm.at[idx_v.at[pl.ds(0, CHUNK)]],
                rows_v.at[b], gsem[b]).wait()

        def scatter_start(c, b):
            pltpu.async_copy(
                rows_v.at[b],
                out_hbm.at[pl.ds(base + c * CHUNK, CHUNK)], ssem[b])

        def scatter_wait(c, b):
            pltpu.make_async_copy(
                rows_v.at[b],
                out_hbm.at[pl.ds(base + c * CHUNK, CHUNK)], ssem[b]).wait()

        gather_start(0, 0)
        gather_start(1, 1)

        gather_wait(0)
        scatter_start(0, 0)
        gather_start(2, 2)

        assert (nchunks - 5) % NBUF == 0
        nsteps = (nchunks - 5) // NBUF

        def full_step(c, b, bp):
            gather_wait(b)
            scatter_start(c, b)
            scatter_wait(c - 1, bp)
            gather_start(c + 2, bp)

        def body(j, carry):
            for p in range(NBUF):
                full_step(1 + j * NBUF + p, (1 + p) % NBUF, p % NBUF)
            return carry

        lax.fori_loop(0, nsteps, body, 0)

        for c in range(nchunks - 4, nchunks - 2):
            full_step(c, c % NBUF, (c - 1) % NBUF)

        for c in range(nchunks - 2, nchunks):
            b = c % NBUF
            gather_wait(b)
            scatter_start(c, b)
        for c in range(nchunks - 3, nchunks):
            scatter_wait(c, c % NBUF)

    return k


def kernel(x, weight):
    B, L = x.shape
    V, D = weight.shape
    idx = x.T.reshape(-1).astype(jnp.int32)
    gather = _make_gather_plane(B * L, D)
    outr = gather(idx, jnp.real(weight))
    outi = gather(idx, jnp.imag(weight))
    out = lax.complex(outr.reshape(L, B, D), outi.reshape(L, B, D))
    return jnp.transpose(out, (1, 0, 2))


# speedup vs baseline: 1.0010x; 1.0010x over previous
import functools

import jax
import jax.numpy as jnp
from jax import lax
from jax.experimental import pallas as pl
from jax.experimental.pallas import tpu as pltpu
from jax.experimental.pallas import tpu_sc as plsc

NC, NS = 2, 16
NW = NC * NS
CHUNK = 128
NBUF = 2


def _make_gather_plane(B, D):
    assert B % (NW * CHUNK) == 0
    bpw = B // NW
    nchunks = bpw // CHUNK

    mesh = plsc.VectorSubcoreMesh(
        core_axis_name="c", subcore_axis_name="s",
        num_cores=NC, num_subcores=NS)

    @functools.partial(
        pl.kernel,
        out_type=jax.ShapeDtypeStruct((B, D), jnp.float32),
        mesh=mesh,
        scratch_types=[
            pltpu.VMEM((bpw,), jnp.int32),
            pltpu.VMEM((NBUF, CHUNK, D), jnp.float32),
        ] + [pltpu.SemaphoreType.DMA] * (2 * NBUF),
    )
    def k(idx_hbm, w_hbm, out_hbm, idx_v, rows_v, *sems):
        gsem = sems[0:NBUF]
        ssem = sems[NBUF:2 * NBUF]
        wid = lax.axis_index("s") * NC + lax.axis_index("c")
        base = wid * bpw
        pltpu.sync_copy(idx_hbm.at[pl.ds(base, bpw)], idx_v)

        def gather_start(c, b):
            pltpu.async_copy(
                w_hbm.at[idx_v.at[pl.ds(c * CHUNK, CHUNK)]],
                rows_v.at[b], gsem[b])

        def gather_wait(b):
            pltpu.make_async_copy(
                w_hbm.at[idx_v.at[pl.ds(0, CHUNK)]],
                rows_v.at[b], gsem[b]).wait()

        def scatter_start(c, b):
            pltpu.async_copy(
                rows_v.at[b],
                out_hbm.at[pl.ds(base + c * CHUNK, CHUNK)], ssem[b])

        def scatter_wait(c, b):
            pltpu.make_async_copy(
                rows_v.at[b],
                out_hbm.at[pl.ds(base + c * CHUNK, CHUNK)], ssem[b]).wait()

        for b in range(NBUF):
            gather_start(b, b)

        def body(j, carry):
            for b in range(NBUF):
                c = j * NBUF + b
                gather_wait(b)
                scatter_start(c, b)
                scatter_wait(c, b)
                gather_start(c + NBUF, b)
            return carry

        lax.fori_loop(0, (nchunks - NBUF) // NBUF, body, 0)

        for b in range(NBUF):
            c = nchunks - NBUF + b
            gather_wait(b)
            scatter_start(c, b)
        for b in range(NBUF):
            c = nchunks - NBUF + b
            scatter_wait(c, b)

    return k


def kernel(x, weight):
    B, L = x.shape
    V, D = weight.shape
    idx = x.T.reshape(-1).astype(jnp.int32)
    gather = _make_gather_plane(B * L, D)
    outr = gather(idx, jnp.real(weight))
    outi = gather(idx, jnp.imag(weight))
    out = lax.complex(outr.reshape(L, B, D), outi.reshape(L, B, D))
    return jnp.transpose(out, (1, 0, 2))
